# Initial kernel scaffold; baseline (speedup 1.0000x reference)
import functools

import jax
import jax.numpy as jnp
from jax import lax
from jax.experimental import pallas as pl
from jax.experimental.pallas import tpu as pltpu
from jax.experimental.pallas import tpu_sc as plsc

BATCH = 16384
NUM_FIELDS = 26
NUM_NUMERICAL = 13
BN_EPS = 1e-5

NC = 2
NS = 16
NW = NC * NS
BPW = BATCH // NW
VECS = BPW // 16
CHUNK = 128
CPF = BPW // CHUNK


def _sc_body(idx_hbm, offs_hbm, table_hbm, out_hbm, idx_v, offs_v, vals_v,
             acc_v, sem):
    c = lax.axis_index("c")
    s = lax.axis_index("s")
    wid = s * NC + c
    base = wid * BPW

    pltpu.sync_copy(idx_hbm.at[:, pl.ds(base, BPW)], idx_v)
    pltpu.sync_copy(offs_hbm, offs_v)

    def _field(f, _):
        off = offs_v[f]

        def _vec(v, _):
            sl = pl.ds(v * 16, 16)
            idx_v[f, sl] = idx_v[f, sl] + off
            return 0

        lax.fori_loop(0, VECS, _vec, 0, unroll=True)
        return 0

    lax.fori_loop(0, NUM_FIELDS, _field, 0)

    def _fire(j, _):
        f = j // CPF
        ch = j - f * CPF
        src = table_hbm.at[idx_v.at[f, pl.ds(ch * CHUNK, CHUNK)]]
        dst = vals_v.at[pl.ds(j * CHUNK, CHUNK)]
        pltpu.make_async_copy(src, dst, sem).start()
        return 0

    lax.fori_loop(0, NUM_FIELDS * CPF, _fire, 0)
    pltpu.make_async_copy(
        table_hbm.at[pl.ds(0, NUM_FIELDS * BPW)], vals_v, sem).wait()

    def _zero(v, _):
        acc_v[pl.ds(v * 16, 16)] = jnp.zeros((16,), jnp.float32)
        return 0

    lax.fori_loop(0, VECS, _zero, 0, unroll=True)

    def _accf(f, _):
        def _vec(v, _):
            sl = pl.ds(v * 16, 16)
            acc_v[sl] = acc_v[sl] + vals_v[pl.ds(f * BPW + v * 16, 16)]
            return 0

        lax.fori_loop(0, VECS, _vec, 0, unroll=True)
        return 0

    lax.fori_loop(0, NUM_FIELDS, _accf, 0)

    pltpu.sync_copy(acc_v, out_hbm.at[pl.ds(base, BPW)])


@jax.jit
def _sc_gather_sum(idx_t, offs, table1d):
    mesh = plsc.VectorSubcoreMesh(core_axis_name="c", subcore_axis_name="s")
    return pl.kernel(
        _sc_body,
        out_type=jax.ShapeDtypeStruct((BATCH,), jnp.float32),
        mesh=mesh,
        scratch_types=[
            pltpu.VMEM((NUM_FIELDS, BPW), jnp.int32),
            pltpu.VMEM((32,), jnp.int32),
            pltpu.VMEM((NUM_FIELDS * BPW,), jnp.float32),
            pltpu.VMEM((BPW,), jnp.float32),
            pltpu.SemaphoreType.DMA,
        ],
    )(idx_t, offs, table1d)


def _tc_body(num_ref, w_ref, bias_ref, gamma_ref, beta_ref, cat_ref, out_ref):
    x = num_ref[...]
    mean = jnp.mean(x, axis=0, keepdims=True)
    var = jnp.mean((x - mean) ** 2, axis=0, keepdims=True)
    xn = (x - mean) * lax.rsqrt(var + BN_EPS) * gamma_ref[...] + beta_ref[...]
    lin = jnp.sum(xn * w_ref[...], axis=1, keepdims=True)
    out_ref[...] = lin + cat_ref[...] + bias_ref[0, 0]


@jax.jit
def _tc_bn_linear(numerical_x, w, bias, gamma, beta, cat_sum):
    return pl.pallas_call(
        _tc_body,
        out_shape=jax.ShapeDtypeStruct((BATCH, 1), jnp.float32),
    )(numerical_x, w, bias, gamma, beta, cat_sum)


def kernel(numerical_x, categorical_x, cat_table, num_weight, bias, bn_gamma,
           bn_beta, field_offsets):
    idx_t = categorical_x.astype(jnp.int32).T
    offs = jnp.zeros((32,), jnp.int32).at[:NUM_FIELDS].set(
        field_offsets.astype(jnp.int32))
    table1d = cat_table.reshape(-1)
    cat_sum = _sc_gather_sum(idx_t, offs, table1d)
    return _tc_bn_linear(
        numerical_x, num_weight, bias.reshape(1, 1),
        bn_gamma.reshape(1, NUM_NUMERICAL), bn_beta.reshape(1, NUM_NUMERICAL),
        cat_sum.reshape(BATCH, 1))


# speedup vs baseline: 1.0157x; 1.0157x over previous
import functools

import jax
import jax.numpy as jnp
from jax import lax
from jax.experimental import pallas as pl
from jax.experimental.pallas import tpu as pltpu
from jax.experimental.pallas import tpu_sc as plsc

BATCH = 16384
NUM_FIELDS = 26
FIELD_VOCAB = 1000000
NUM_NUMERICAL = 13
BN_EPS = 1e-5

NC = 2
NS = 16
NW = NC * NS
BPW = BATCH // NW
VECS = BPW // 16
CHUNK = 128
CPF = BPW // CHUNK


def _sc_body(idx_hbm, table_hbm, out_hbm, idx_v, vals_v, acc_v, sem):
    c = lax.axis_index("c")
    s = lax.axis_index("s")
    wid = s * NC + c
    base = wid * BPW

    pltpu.sync_copy(idx_hbm.at[:, pl.ds(base, BPW)], idx_v)

    def _field(f, _):
        off = f * FIELD_VOCAB

        def _vec(v, _):
            sl = pl.ds(v * 16, 16)
            idx_v[f, sl] = idx_v[f, sl] + off
            return 0

        lax.fori_loop(0, VECS, _vec, 0, unroll=True)
        return 0

    lax.fori_loop(0, NUM_FIELDS, _field, 0)

    def _fire(j, _):
        f = j // CPF
        ch = j - f * CPF
        src = table_hbm.at[idx_v.at[f, pl.ds(ch * CHUNK, CHUNK)]]
        dst = vals_v.at[pl.ds(j * CHUNK, CHUNK)]
        pltpu.make_async_copy(src, dst, sem).start()
        return 0

    lax.fori_loop(0, NUM_FIELDS * CPF, _fire, 0)
    pltpu.make_async_copy(
        table_hbm.at[pl.ds(0, NUM_FIELDS * BPW)], vals_v, sem).wait()

    def _zero(v, _):
        acc_v[pl.ds(v * 16, 16)] = jnp.zeros((16,), jnp.float32)
        return 0

    lax.fori_loop(0, VECS, _zero, 0, unroll=True)

    def _accf(f, _):
        def _vec(v, _):
            sl = pl.ds(v * 16, 16)
            acc_v[sl] = acc_v[sl] + vals_v[pl.ds(f * BPW + v * 16, 16)]
            return 0

        lax.fori_loop(0, VECS, _vec, 0, unroll=True)
        return 0

    lax.fori_loop(0, NUM_FIELDS, _accf, 0)

    pltpu.sync_copy(acc_v, out_hbm.at[pl.ds(base, BPW)])


@jax.jit
def _sc_gather_sum(idx_t, table1d):
    mesh = plsc.VectorSubcoreMesh(core_axis_name="c", subcore_axis_name="s")
    return pl.kernel(
        _sc_body,
        out_type=jax.ShapeDtypeStruct((BATCH,), jnp.float32),
        mesh=mesh,
        scratch_types=[
            pltpu.VMEM((NUM_FIELDS, BPW), jnp.int32),
            pltpu.VMEM((NUM_FIELDS * BPW,), jnp.float32),
            pltpu.VMEM((BPW,), jnp.float32),
            pltpu.SemaphoreType.DMA,
        ],
    )(idx_t, table1d)


def _tc_body(num_ref, w_ref, bias_ref, gamma_ref, beta_ref, cat_ref, out_ref):
    x = num_ref[...]
    mean = jnp.mean(x, axis=0, keepdims=True)
    var = jnp.mean((x - mean) ** 2, axis=0, keepdims=True)
    xn = (x - mean) * lax.rsqrt(var + BN_EPS) * gamma_ref[...] + beta_ref[...]
    lin = jnp.sum(xn * w_ref[...], axis=1, keepdims=True)
    out_ref[...] = lin + cat_ref[...] + bias_ref[0, 0]


@jax.jit
def _tc_bn_linear(numerical_x, w, bias, gamma, beta, cat_sum):
    return pl.pallas_call(
        _tc_body,
        out_shape=jax.ShapeDtypeStruct((BATCH, 1), jnp.float32),
    )(numerical_x, w, bias, gamma, beta, cat_sum)


def kernel(numerical_x, categorical_x, cat_table, num_weight, bias, bn_gamma,
           bn_beta, field_offsets):
    idx_t = categorical_x.astype(jnp.int32).T
    table1d = cat_table.reshape(-1)
    cat_sum = _sc_gather_sum(idx_t, table1d)
    return _tc_bn_linear(
        numerical_x, num_weight, bias.reshape(1, 1),
        bn_gamma.reshape(1, NUM_NUMERICAL), bn_beta.reshape(1, NUM_NUMERICAL),
        cat_sum.reshape(BATCH, 1))
